# baseline (speedup 1.0000x reference)
import functools
import math

import jax
import jax.numpy as jnp
import numpy as np
from jax.experimental import pallas as pl
from jax.experimental.pallas import tpu as pltpu


def _encoder_block(lens_sm, emb_ref, times_ref, freq_ref, out_ref,
                   *, n, e, scale, smax):
    b = pl.program_id(0)
    T = emb_ref.shape[1]

    t0 = pl.program_id(1) * emb_ref.shape[1]
    tv = times_ref[b, pl.ds(t0, emb_ref.shape[1]), :]
    idxf = jnp.clip(jnp.round(tv * 10.0), 0.0, float(smax))
    angle = idxf * freq_ref[...]
    lane = jax.lax.broadcasted_iota(jnp.int32, angle.shape, 1)
    row = jnp.where(lane % 2 == 0, jnp.sin(angle), jnp.cos(angle))

    seqlen = lens_sm[b]
    tvec = t0 + jax.lax.broadcasted_iota(jnp.int32, (T, 1), 0)
    valid = (tvec < seqlen).astype(jnp.float32)
    sin_embed = row * valid

    for i in range(n):
        sl = slice(i * e, (i + 1) * e)
        out_ref[0, :, sl] = emb_ref[0, :, sl] * scale + sin_embed


def kernel(embeddings, times, sequence_lengths, sinusoidal_table):
    B, T, N, E = embeddings.shape
    S = sinusoidal_table.shape[0]
    scale = math.sqrt(E)
    tbl = 40

    div = np.exp(np.arange(0, E, 2, dtype=np.float32) *
                 (-math.log(10000.0) / E))
    freq = jnp.asarray(np.repeat(div, 2).reshape(1, E))

    grid_spec = pltpu.PrefetchScalarGridSpec(
        num_scalar_prefetch=1,
        grid=(B, T // tbl),
        in_specs=[
            pl.BlockSpec((1, tbl, N * E), lambda b, t, *_: (b, t, 0)),
            pl.BlockSpec((B, T, 1), lambda b, t, *_: (0, 0, 0)),
            pl.BlockSpec((1, E), lambda b, t, *_: (0, 0)),
        ],
        out_specs=pl.BlockSpec((1, tbl, N * E), lambda b, t, *_: (b, t, 0)),
    )

    out = pl.pallas_call(
        functools.partial(_encoder_block, n=N, e=E, scale=scale, smax=S - 1),
        grid_spec=grid_spec,
        out_shape=jax.ShapeDtypeStruct((B, T, N * E), jnp.float32),
    )(sequence_lengths.astype(jnp.int32), embeddings.reshape(B, T, N * E),
      times.reshape(B, T, 1), freq)
    return out.reshape(B, T, N, E)


# speedup vs baseline: 1.4379x; 1.4379x over previous
import functools
import math

import jax
import jax.numpy as jnp
import numpy as np
from jax.experimental import pallas as pl
from jax.experimental.pallas import tpu as pltpu


def _encoder_block(lens_sm, emb_ref, times_ref, freq_ref, out_ref,
                   *, bb, n, e, scale, smax):
    b0 = pl.program_id(0) * bb
    T = emb_ref.shape[1]

    for kb in range(bb):
        b = b0 + kb
        tv = times_ref[b]
        idxf = jnp.clip(jnp.round(tv * 10.0), 0.0, float(smax))
        angle = idxf * freq_ref[...]
        lane = jax.lax.broadcasted_iota(jnp.int32, angle.shape, 1)
        row = jnp.where(lane % 2 == 0, jnp.sin(angle), jnp.cos(angle))

        seqlen = lens_sm[b]
        tvec = jax.lax.broadcasted_iota(jnp.int32, (T, 1), 0)
        valid = (tvec < seqlen).astype(jnp.float32)
        sin_embed = row * valid

        for i in range(n):
            sl = slice(i * e, (i + 1) * e)
            out_ref[kb, :, sl] = emb_ref[kb, :, sl] * scale + sin_embed


def kernel(embeddings, times, sequence_lengths, sinusoidal_table):
    B, T, N, E = embeddings.shape
    S = sinusoidal_table.shape[0]
    scale = math.sqrt(E)
    bb = 2

    div = np.exp(np.arange(0, E, 2, dtype=np.float32) *
                 (-math.log(10000.0) / E))
    freq = jnp.asarray(np.repeat(div, 2).reshape(1, E))

    grid_spec = pltpu.PrefetchScalarGridSpec(
        num_scalar_prefetch=1,
        grid=(B // bb,),
        in_specs=[
            pl.BlockSpec((bb, T, N * E), lambda b, *_: (b, 0, 0)),
            pl.BlockSpec((B, T, 1), lambda b, *_: (0, 0, 0)),
            pl.BlockSpec((1, E), lambda b, *_: (0, 0)),
        ],
        out_specs=pl.BlockSpec((bb, T, N * E), lambda b, *_: (b, 0, 0)),
    )

    out = pl.pallas_call(
        functools.partial(_encoder_block, bb=bb, n=N, e=E, scale=scale,
                          smax=S - 1),
        grid_spec=grid_spec,
        out_shape=jax.ShapeDtypeStruct((B, T, N * E), jnp.float32),
    )(sequence_lengths.astype(jnp.int32), embeddings.reshape(B, T, N * E),
      times.reshape(B, T, 1), freq)
    return out.reshape(B, T, N, E)


# speedup vs baseline: 1.4417x; 1.0027x over previous
import functools
import math

import jax
import jax.numpy as jnp
import numpy as np
from jax.experimental import pallas as pl
from jax.experimental.pallas import tpu as pltpu


def _encoder_block(lens_sm, emb_ref, times_ref, freq_ref, out_ref,
                   *, bb, n, e, scale, smax):
    b0 = pl.program_id(0) * bb
    T = emb_ref.shape[1]

    for kb in range(bb):
        b = b0 + kb
        tv = times_ref[b]
        idxf = jnp.clip(jnp.round(tv * 10.0), 0.0, float(smax))
        angle = idxf * freq_ref[...]
        lane = jax.lax.broadcasted_iota(jnp.int32, angle.shape, 1)
        row = jnp.where(lane % 2 == 0, jnp.sin(angle), jnp.cos(angle))

        seqlen = lens_sm[b]
        tvec = jax.lax.broadcasted_iota(jnp.int32, (T, 1), 0)
        valid = (tvec < seqlen).astype(jnp.float32)
        sin_embed = row * valid

        for i in range(n):
            sl = slice(i * e, (i + 1) * e)
            out_ref[kb, :, sl] = emb_ref[kb, :, sl] * scale + sin_embed


def kernel(embeddings, times, sequence_lengths, sinusoidal_table):
    B, T, N, E = embeddings.shape
    S = sinusoidal_table.shape[0]
    scale = math.sqrt(E)
    bb = 4

    div = np.exp(np.arange(0, E, 2, dtype=np.float32) *
                 (-math.log(10000.0) / E))
    freq = jnp.asarray(np.repeat(div, 2).reshape(1, E))

    grid_spec = pltpu.PrefetchScalarGridSpec(
        num_scalar_prefetch=1,
        grid=(B // bb,),
        in_specs=[
            pl.BlockSpec((bb, T, N * E), lambda b, *_: (b, 0, 0)),
            pl.BlockSpec((B, T, 1), lambda b, *_: (0, 0, 0)),
            pl.BlockSpec((1, E), lambda b, *_: (0, 0)),
        ],
        out_specs=pl.BlockSpec((bb, T, N * E), lambda b, *_: (b, 0, 0)),
    )

    out = pl.pallas_call(
        functools.partial(_encoder_block, bb=bb, n=N, e=E, scale=scale,
                          smax=S - 1),
        grid_spec=grid_spec,
        out_shape=jax.ShapeDtypeStruct((B, T, N * E), jnp.float32),
    )(sequence_lengths.astype(jnp.int32), embeddings.reshape(B, T, N * E),
      times.reshape(B, T, 1), freq)
    return out.reshape(B, T, N, E)


# speedup vs baseline: 1.4514x; 1.0067x over previous
import functools
import math

import jax
import jax.numpy as jnp
import numpy as np
from jax.experimental import pallas as pl
from jax.experimental.pallas import tpu as pltpu

_NBUF = 4


def _encoder_pipe(emb_ref, times_ref, lens_ref, freq_ref, out_ref,
                  in_buf, out_buf, in_sems, out_sems,
                  *, nb, n, e, scale, smax):
    T = in_buf.shape[1]

    def in_copy(i, buf):
        return pltpu.make_async_copy(emb_ref.at[i], in_buf.at[buf],
                                     in_sems.at[buf])

    def out_copy(i, buf):
        return pltpu.make_async_copy(out_buf.at[buf], out_ref.at[i],
                                     out_sems.at[buf])

    for j in range(_NBUF):
        in_copy(j, j).start()

    def step(i, carry):
        buf = jax.lax.rem(i, _NBUF)
        in_copy(i, buf).wait()

        @pl.when(i >= _NBUF)
        def _():
            out_copy(i - _NBUF, buf).wait()

        tv = times_ref[i]
        idxf = jnp.clip(jnp.round(tv * 10.0), 0.0, float(smax))
        angle = idxf * freq_ref[...]
        lane = jax.lax.broadcasted_iota(jnp.int32, angle.shape, 1)
        row = jnp.where(lane % 2 == 0, jnp.sin(angle), jnp.cos(angle))

        seqlen = lens_ref[i]
        tvec = jax.lax.broadcasted_iota(jnp.int32, (T, 1), 0)
        valid = (tvec < seqlen).astype(jnp.float32)
        sin_embed = row * valid

        for k in range(n):
            sl = slice(k * e, (k + 1) * e)
            out_buf[buf, :, sl] = in_buf[buf, :, sl] * scale + sin_embed

        out_copy(i, buf).start()

        @pl.when(i + _NBUF < nb)
        def _():
            in_copy(i + _NBUF, buf).start()

        return carry

    jax.lax.fori_loop(0, nb, step, 0)

    for j in range(_NBUF):
        i = nb - _NBUF + j
        out_copy(i, jax.lax.rem(jnp.int32(i), _NBUF)).wait()


def kernel(embeddings, times, sequence_lengths, sinusoidal_table):
    B, T, N, E = embeddings.shape
    S = sinusoidal_table.shape[0]
    scale = math.sqrt(E)

    div = np.exp(np.arange(0, E, 2, dtype=np.float32) *
                 (-math.log(10000.0) / E))
    freq = jnp.asarray(np.repeat(div, 2).reshape(1, E))

    out = pl.pallas_call(
        functools.partial(_encoder_pipe, nb=B, n=N, e=E, scale=scale,
                          smax=S - 1),
        in_specs=[
            pl.BlockSpec(memory_space=pl.ANY),
            pl.BlockSpec(memory_space=pltpu.VMEM),
            pl.BlockSpec(memory_space=pltpu.SMEM),
            pl.BlockSpec(memory_space=pltpu.VMEM),
        ],
        out_specs=pl.BlockSpec(memory_space=pl.ANY),
        out_shape=jax.ShapeDtypeStruct((B, T, N * E), jnp.float32),
        scratch_shapes=[
            pltpu.VMEM((_NBUF, T, N * E), jnp.float32),
            pltpu.VMEM((_NBUF, T, N * E), jnp.float32),
            pltpu.SemaphoreType.DMA((_NBUF,)),
            pltpu.SemaphoreType.DMA((_NBUF,)),
        ],
    )(embeddings.reshape(B, T, N * E), times.reshape(B, T, 1),
      sequence_lengths.astype(jnp.int32), freq)
    return out.reshape(B, T, N, E)
